# Initial kernel scaffold; baseline (speedup 1.0000x reference)
import jax
import jax.numpy as jnp
from jax.experimental import pallas as pl


def kernel(token_positions, positional_embeddings):
    raise NotImplementedError("write your pallas kernel here")


# speedup vs baseline: 1.5785x; 1.5785x over previous
import functools

import jax
import jax.numpy as jnp
from jax import lax
from jax.experimental import pallas as pl
from jax.experimental.pallas import tpu as pltpu
from jax.experimental.pallas import tpu_sc as plsc

_INFO = plsc.get_sparse_core_info()
_NC = _INFO.num_cores
_NS = _INFO.num_subcores
_NW = _NC * _NS


def _gather_kernel(n, d, b_per_w, ch):
    n_chunks = b_per_w // ch
    mesh = plsc.VectorSubcoreMesh(core_axis_name="c", subcore_axis_name="s")

    @functools.partial(
        pl.kernel,
        mesh=mesh,
        out_type=jax.ShapeDtypeStruct((n, d), jnp.float32),
        scratch_types=[
            pltpu.VMEM((n_chunks, ch), jnp.int32),
            pltpu.VMEM((ch, d), jnp.float32),
            pltpu.VMEM((ch, d), jnp.float32),
            pltpu.SemaphoreType.DMA,
            pltpu.SemaphoreType.DMA,
            pltpu.SemaphoreType.DMA,
        ],
    )
    def k(idx_hbm, table_hbm, out_hbm, idx_v, buf0, buf1, gsem, osem0, osem1):
        wid = lax.axis_index("s") * _NC + lax.axis_index("c")
        base = wid * b_per_w
        pltpu.sync_copy(idx_hbm.at[wid], idx_v)
        idx_w = idx_v

        bufs = (buf0, buf1)
        osems = (osem0, osem1)

        def start_gather(c, slot):
            pltpu.async_copy(table_hbm.at[idx_w.at[c]], bufs[slot], gsem)

        def wait_gather(c, slot):
            pltpu.make_async_copy(
                table_hbm.at[idx_w.at[c]], bufs[slot], gsem
            ).wait()

        def out_ref(c):
            return out_hbm.at[pl.ds(base + c * ch, ch), :]

        def start_out(c, slot):
            pltpu.async_copy(bufs[slot], out_ref(c), osems[slot])

        def wait_out(c, slot):
            pltpu.make_async_copy(bufs[slot], out_ref(c), osems[slot]).wait()

        start_gather(0, 0)
        wait_gather(0, 0)
        start_out(0, 0)
        start_gather(1, 1)

        @pl.loop(1, n_chunks - 1, step=2)
        def _(c):
            for j in range(2):
                slot = (1 + j) % 2
                cc = c + j
                wait_gather(cc, slot)
                start_out(cc, slot)
                wait_out(cc - 1, 1 - slot)
                start_gather(cc + 1, 1 - slot)

        last = n_chunks - 1
        wait_gather(last, last % 2)
        start_out(last, last % 2)
        wait_out(last - 1, (last - 1) % 2)
        wait_out(last, last % 2)

    return k


@jax.jit
def kernel(token_positions, positional_embeddings):
    b, s = token_positions.shape
    v, d = positional_embeddings.shape
    n = b * s
    b_per_w = n // _NW
    ch = 16
    idx = token_positions.astype(jnp.int32).reshape(_NW, b_per_w // ch, ch)
    out = _gather_kernel(n, d, b_per_w, ch)(idx, positional_embeddings)
    return out.reshape(b, s, d)


# speedup vs baseline: 1.6014x; 1.0146x over previous
import functools

import jax
import jax.numpy as jnp
from jax import lax
from jax.experimental import pallas as pl
from jax.experimental.pallas import tpu as pltpu
from jax.experimental.pallas import tpu_sc as plsc

_INFO = plsc.get_sparse_core_info()
_NC = _INFO.num_cores
_NS = _INFO.num_subcores
_NW = _NC * _NS

_CH = 16
_NBUF = 3


def _gather_kernel(n, d, b_per_w, ch, nbuf):
    n_chunks = b_per_w // ch
    mesh = plsc.VectorSubcoreMesh(core_axis_name="c", subcore_axis_name="s")

    @functools.partial(
        pl.kernel,
        mesh=mesh,
        out_type=jax.ShapeDtypeStruct((n, d), jnp.float32),
        scratch_types=[
            pltpu.VMEM((n_chunks, ch), jnp.int32),
        ]
        + [pltpu.VMEM((ch, d), jnp.float32) for _ in range(nbuf)]
        + [pltpu.SemaphoreType.DMA for _ in range(2 * nbuf)],
    )
    def k(idx_hbm, table_hbm, out_hbm, idx_v, *rest):
        bufs = rest[:nbuf]
        gsems = rest[nbuf : 2 * nbuf]
        osems = rest[2 * nbuf :]

        wid = lax.axis_index("s") * _NC + lax.axis_index("c")
        base = wid * b_per_w
        pltpu.sync_copy(idx_hbm.at[wid], idx_v)

        def start_gather(c, slot):
            pltpu.async_copy(table_hbm.at[idx_v.at[c]], bufs[slot], gsems[slot])

        def wait_gather(c, slot):
            pltpu.make_async_copy(
                table_hbm.at[idx_v.at[c]], bufs[slot], gsems[slot]
            ).wait()

        def out_ref(c):
            return out_hbm.at[pl.ds(base + c * ch, ch), :]

        def start_out(c, slot):
            pltpu.async_copy(bufs[slot], out_ref(c), osems[slot])

        def wait_out(c, slot):
            pltpu.make_async_copy(bufs[slot], out_ref(c), osems[slot]).wait()

        def step(c, slot, do_outwait, do_gather):
            wait_gather(c, slot)
            start_out(c, slot)
            if do_gather:
                if do_outwait:
                    wait_out(c - 1, (slot - 1) % nbuf)
                start_gather(c + nbuf - 1, (slot - 1) % nbuf)

        for j in range(nbuf - 1):
            start_gather(j, j)
        step(0, 0, False, True)

        n_mid = n_chunks - nbuf
        n_loop = (n_mid // nbuf) * nbuf

        @pl.loop(1, 1 + n_loop, step=nbuf)
        def _(c):
            for j in range(nbuf):
                step(c + j, (1 + j) % nbuf, True, True)

        for c in range(1 + n_loop, n_chunks - nbuf + 1):
            step(c, c % nbuf, True, True)
        for c in range(n_chunks - nbuf + 1, n_chunks):
            step(c, c % nbuf, False, False)
        for c in range(n_chunks - nbuf, n_chunks):
            wait_out(c, c % nbuf)

    return k


@jax.jit
def kernel(token_positions, positional_embeddings):
    b, s = token_positions.shape
    v, d = positional_embeddings.shape
    n = b * s
    b_per_w = n // _NW
    idx = token_positions.astype(jnp.int32).reshape(_NW, b_per_w // _CH, _CH)
    out = _gather_kernel(n, d, b_per_w, _CH, _NBUF)(idx, positional_embeddings)
    return out.reshape(b, s, d)
